# baseline (speedup 1.0000x reference)
import functools

import jax
import jax.numpy as jnp
from jax import lax
from jax.experimental import pallas as pl
from jax.experimental.pallas import tpu as pltpu
from jax.experimental.pallas import tpu_sc as plsc

F = 8
P = 2048
N = 2048
HALF = N // 2
RBLK = 256

_info = plsc.get_sparse_core_info()
NC = _info.num_cores
NS = _info.num_subcores
L = _info.num_lanes
NW = NC * NS
ROWS_PER_W = N // NW
CHUNK = 8
NCHUNK = ROWS_PER_W // CHUNK


def _combine_kernel(zst_ref, fcw_ref, fcb_ref, a_ref, m_ref):
    prod = zst_ref[...] * fcw_ref[...]
    logits = jnp.sum(prod, axis=1, keepdims=True) + fcb_ref[...]
    logits = logits - jnp.max(logits, axis=0, keepdims=True)
    e = jnp.exp(logits)
    w = e / jnp.sum(e, axis=0, keepdims=True)
    hw = 0.5 * w
    a = a_ref[...]
    t = jnp.sum(hw[:, :, None] * jnp.tanh(0.5 * a), axis=0) + jnp.sum(hw)
    u = jnp.minimum((t * 65535.0 + 0.5).astype(jnp.int32), 65535)
    m_ref[...] = u[:, :HALF] | (u[:, HALF:] << 16)


def _combine(zst, fc_w, fc_b2, factors):
    return pl.pallas_call(
        _combine_kernel,
        grid=(P // RBLK,),
        in_specs=[
            pl.BlockSpec((F, zst.shape[1]), lambda i: (0, 0)),
            pl.BlockSpec((1, fc_w.shape[1]), lambda i: (0, 0)),
            pl.BlockSpec((1, 1), lambda i: (0, 0)),
            pl.BlockSpec((F, RBLK, N), lambda i: (0, i, 0)),
        ],
        out_specs=pl.BlockSpec((RBLK, HALF), lambda i: (i, 0)),
        out_shape=jax.ShapeDtypeStruct((P, HALF), jnp.int32),
    )(zst, fc_w, fc_b2, factors)


@functools.partial(
    pl.kernel,
    mesh=plsc.VectorSubcoreMesh(core_axis_name="c", subcore_axis_name="s"),
    out_type=jax.ShapeDtypeStruct((N, N), jnp.float32),
    compiler_params=pltpu.CompilerParams(
        use_tc_tiling_on_sc=True, needs_layout_passes=False
    ),
    scratch_types=[
        pltpu.VMEM((N,), jnp.float32),
        pltpu.VMEM((N,), jnp.int32),
        pltpu.VMEM((CHUNK, HALF), jnp.int32),
        pltpu.VMEM((CHUNK, HALF), jnp.int32),
        pltpu.VMEM((CHUNK, N), jnp.float32),
        pltpu.VMEM((CHUNK, N), jnp.float32),
        pltpu.SemaphoreType.DMA,
        pltpu.SemaphoreType.DMA,
    ],
)
def _gather_kernel(m_hbm, vs_hbm, out_hbm, vs_v, idx_v, rows0, rows1,
                   outb0, outb1, sem_in, sem_out):
    wid = lax.axis_index("s") * NC + lax.axis_index("c")
    base = wid * ROWS_PER_W
    base16 = wid * (ROWS_PER_W // L)

    def mk_idx(k, carry):
        v = vs_v[pl.ds(k * L, L)]
        i = (v * jnp.float32(P)).astype(jnp.int32)
        idx_v[pl.ds(k * L, L)] = jnp.minimum(jnp.maximum(i, 0), P - 1)
        return carry

    rows = [rows0, rows1]
    outb = [outb0, outb1]

    def start_row_gather(c):
        cp = pltpu.make_async_copy(
            m_hbm.at[idx_v.at[pl.ds(base + c * CHUNK, CHUNK)]],
            rows[c % 2],
            sem_in,
        )
        cp.start()
        return cp

    pltpu.sync_copy(vs_hbm.at[pl.ds(base, ROWS_PER_W)],
                    vs_v.at[pl.ds(base, ROWS_PER_W)])
    lax.fori_loop(base16, base16 + ROWS_PER_W // L, mk_idx, 0)
    in_cp = {0: start_row_gather(0)}

    pltpu.sync_copy(vs_hbm, vs_v)
    lax.fori_loop(0, base16, mk_idx, 0)
    lax.fori_loop(base16 + ROWS_PER_W // L, N // L, mk_idx, 0)
    out_cp = [None, None]

    for c in range(NCHUNK):
        buf = c % 2
        in_cp[buf].wait()
        if c + 1 < NCHUNK:
            in_cp[(c + 1) % 2] = start_row_gather(c + 1)
        if out_cp[buf] is not None:
            out_cp[buf].wait()

        @plsc.parallel_loop(0, N // L, 1, unroll=4)
        def col_gather(k):
            cvec = idx_v[pl.ds(k * L, L)]
            cword = cvec & (HALF - 1)
            shift = jnp.where(cvec < HALF, 0, 16)
            for r in range(CHUNK):
                rvec = jnp.full((L,), r, jnp.int32)
                word = plsc.load_gather(rows[buf], [rvec, cword])
                u = jax.lax.shift_right_logical(word, shift) & 0xFFFF
                vals = u.astype(jnp.float32) * jnp.float32(1.0 / 65535.0)
                outb[buf][r, pl.ds(k * L, L)] = vals

        cp = pltpu.make_async_copy(
            outb[buf],
            out_hbm.at[pl.ds(base + c * CHUNK, CHUNK)],
            sem_out,
        )
        cp.start()
        out_cp[buf] = cp

    out_cp[0].wait()
    out_cp[1].wait()


def kernel(zs, vs, factors_graphon, fc_w, fc_b):
    zst = zs.T
    fc_b2 = fc_b.reshape(1, 1)
    m = _combine(zst, fc_w, fc_b2, factors_graphon)
    return _gather_kernel(m, vs)


# speedup vs baseline: 1.0052x; 1.0052x over previous
import functools

import jax
import jax.numpy as jnp
from jax import lax
from jax.experimental import pallas as pl
from jax.experimental.pallas import tpu as pltpu
from jax.experimental.pallas import tpu_sc as plsc

F = 8
P = 2048
N = 2048
HALF = N // 2
RBLK = 256

_info = plsc.get_sparse_core_info()
NC = _info.num_cores
NS = _info.num_subcores
L = _info.num_lanes
NW = NC * NS
ROWS_PER_W = N // NW
CHUNK = 8
NCHUNK = ROWS_PER_W // CHUNK


def _combine_kernel(zst_ref, fcw_ref, fcb_ref, a_ref, m_ref):
    prod = zst_ref[...] * fcw_ref[...]
    logits = jnp.sum(prod, axis=1, keepdims=True) + fcb_ref[...]
    logits = logits - jnp.max(logits, axis=0, keepdims=True)
    e = jnp.exp(logits)
    w = e / jnp.sum(e, axis=0, keepdims=True)
    hw = 0.5 * w
    a = a_ref[...]
    t = jnp.sum(hw[:, :, None] * jnp.tanh(0.5 * a), axis=0) + jnp.sum(hw)
    u = jnp.minimum((t * 65535.0 + 0.5).astype(jnp.int32), 65535)
    m_ref[...] = u[:, :HALF] | (u[:, HALF:] << 16)


def _combine(zst, fc_w, fc_b2, factors):
    return pl.pallas_call(
        _combine_kernel,
        grid=(P // RBLK,),
        in_specs=[
            pl.BlockSpec((F, zst.shape[1]), lambda i: (0, 0)),
            pl.BlockSpec((1, fc_w.shape[1]), lambda i: (0, 0)),
            pl.BlockSpec((1, 1), lambda i: (0, 0)),
            pl.BlockSpec((F, RBLK, N), lambda i: (0, i, 0)),
        ],
        out_specs=pl.BlockSpec((RBLK, HALF), lambda i: (i, 0)),
        out_shape=jax.ShapeDtypeStruct((P, HALF), jnp.int32),
    )(zst, fc_w, fc_b2, factors)


@functools.partial(
    pl.kernel,
    mesh=plsc.VectorSubcoreMesh(core_axis_name="c", subcore_axis_name="s"),
    out_type=jax.ShapeDtypeStruct((N, N), jnp.float32),
    compiler_params=pltpu.CompilerParams(
        use_tc_tiling_on_sc=True, needs_layout_passes=False
    ),
    scratch_types=[
        pltpu.VMEM((N,), jnp.float32),
        pltpu.VMEM((N,), jnp.int32),
        pltpu.VMEM((CHUNK, HALF), jnp.int32),
        pltpu.VMEM((CHUNK, HALF), jnp.int32),
        pltpu.VMEM((CHUNK, N), jnp.float32),
        pltpu.VMEM((CHUNK, N), jnp.float32),
        pltpu.SemaphoreType.DMA,
        pltpu.SemaphoreType.DMA,
    ],
)
def _gather_kernel(m_hbm, vs_hbm, out_hbm, vs_v, idx_v, rows0, rows1,
                   outb0, outb1, sem_in, sem_out):
    wid = lax.axis_index("s") * NC + lax.axis_index("c")
    base = wid * ROWS_PER_W

    pltpu.sync_copy(vs_hbm, vs_v)

    def mk_idx(k, carry):
        v = vs_v[pl.ds(k * L, L)]
        i = (v * jnp.float32(P)).astype(jnp.int32)
        idx_v[pl.ds(k * L, L)] = jnp.minimum(jnp.maximum(i, 0), P - 1)
        return carry

    lax.fori_loop(0, N // L, mk_idx, 0)

    rows = [rows0, rows1]
    outb = [outb0, outb1]

    def start_row_gather(c):
        cp = pltpu.make_async_copy(
            m_hbm.at[idx_v.at[pl.ds(base + c * CHUNK, CHUNK)]],
            rows[c % 2],
            sem_in,
        )
        cp.start()
        return cp

    in_cp = {0: start_row_gather(0)}
    out_cp = [None, None]

    for c in range(NCHUNK):
        buf = c % 2
        in_cp[buf].wait()
        if c + 1 < NCHUNK:
            in_cp[(c + 1) % 2] = start_row_gather(c + 1)
        if out_cp[buf] is not None:
            out_cp[buf].wait()

        @plsc.parallel_loop(0, N // L, 1, unroll=4)
        def col_gather(k):
            cvec = idx_v[pl.ds(k * L, L)]
            cword = cvec & (HALF - 1)
            shift = jnp.where(cvec < HALF, 0, 16)
            for r in range(CHUNK):
                rvec = jnp.full((L,), r, jnp.int32)
                word = plsc.load_gather(rows[buf], [rvec, cword])
                u = jax.lax.shift_right_logical(word, shift) & 0xFFFF
                vals = u.astype(jnp.float32) * jnp.float32(1.0 / 65535.0)
                outb[buf][r, pl.ds(k * L, L)] = vals

        cp = pltpu.make_async_copy(
            outb[buf],
            out_hbm.at[pl.ds(base + c * CHUNK, CHUNK)],
            sem_out,
        )
        cp.start()
        out_cp[buf] = cp

    out_cp[0].wait()
    out_cp[1].wait()


def kernel(zs, vs, factors_graphon, fc_w, fc_b):
    zst = zs.T
    fc_b2 = fc_b.reshape(1, 1)
    m = _combine(zst, fc_w, fc_b2, factors_graphon)
    return _gather_kernel(m, vs)
